# Initial kernel scaffold; baseline (speedup 1.0000x reference)
import jax
import jax.numpy as jnp
from jax.experimental import pallas as pl


def kernel(x, W1, b1, W2, b2):
    raise NotImplementedError("write your pallas kernel here")


# speedup vs baseline: 5.0456x; 5.0456x over previous
import functools

import jax
import jax.numpy as jnp
from jax import lax
from jax.experimental import pallas as pl
from jax.experimental.pallas import tpu as pltpu
from jax.experimental.pallas import tpu_sc as plsc

N = 100000
D = 256
LANES = 16
NCHUNK = D // LANES

BLK = 125
NBLK = N // BLK
NTREE = 400
NWORKERS = 32
LAST_TREE_R0 = 49875

TC_R = 1000


def _sc_agg_body(h_hbm, out_hbm, par_v, ch_v, out_v):
    wid = lax.axis_index("s") * 2 + lax.axis_index("c")

    def zero_row(ref, row):
        for c in range(NCHUNK):
            ref[row, pl.ds(c * LANES, LANES)] = jnp.zeros((LANES,), jnp.float32)

    zero_row(par_v, 63)

    def load_parent_slab(r0):
        p0 = jnp.maximum((r0 - 1) >> 1, 0)
        pltpu.sync_copy(h_hbm.at[pl.ds(p0, 63)], par_v.at[pl.ds(0, 63)])
        return p0

    def tree_block(t, _):
        bi = t * NWORKERS + wid
        r0 = bi * BLK
        p0 = load_parent_slab(r0)
        c0 = 2 * r0 + 1
        pltpu.sync_copy(h_hbm.at[pl.ds(c0, 249)], ch_v.at[pl.ds(0, 249)])
        last_src = jnp.minimum(c0 + 249, N - 1)
        pltpu.sync_copy(h_hbm.at[pl.ds(last_src, 1)], ch_v.at[pl.ds(249, 1)])
        scale = jnp.where(r0 == LAST_TREE_R0, 0.0, 1.0).astype(jnp.float32)
        for c in range(NCHUNK):
            sl = pl.ds(c * LANES, LANES)
            ch_v[249, sl] = ch_v[249, sl] * scale

        def node(k, _):
            g = r0 + k
            pk = jnp.where(g == 0, 63, ((g - 1) >> 1) - p0)
            for c in range(NCHUNK):
                sl = pl.ds(c * LANES, LANES)
                out_v[k, sl] = par_v[pk, sl] + ch_v[2 * k, sl] + ch_v[2 * k + 1, sl]
            return 0

        lax.fori_loop(0, BLK, node, 0)
        pltpu.sync_copy(out_v, out_hbm.at[pl.ds(r0, BLK)])
        return 0

    def leaf_block(t, _):
        bi = NTREE + t * NWORKERS + wid
        r0 = bi * BLK
        p0 = load_parent_slab(r0)

        def node(k, _):
            pk = ((r0 + k - 1) >> 1) - p0
            for c in range(NCHUNK):
                sl = pl.ds(c * LANES, LANES)
                out_v[k, sl] = par_v[pk, sl]
            return 0

        lax.fori_loop(0, BLK, node, 0)
        pltpu.sync_copy(out_v, out_hbm.at[pl.ds(r0, BLK)])
        return 0

    n_tree = jnp.where(wid < NTREE % NWORKERS, (NTREE // NWORKERS) + 1,
                       NTREE // NWORKERS)
    n_leaf = jnp.where(wid < (NBLK - NTREE) % NWORKERS,
                       ((NBLK - NTREE) // NWORKERS) + 1,
                       (NBLK - NTREE) // NWORKERS)
    lax.fori_loop(0, n_tree, tree_block, 0)
    lax.fori_loop(0, n_leaf, leaf_block, 0)


@jax.jit
def _sc_agg(h):
    mesh = plsc.VectorSubcoreMesh(core_axis_name="c", subcore_axis_name="s")
    return pl.kernel(
        _sc_agg_body,
        out_type=jax.ShapeDtypeStruct((N, D), jnp.float32),
        mesh=mesh,
        scratch_types=[
            pltpu.VMEM((64, D), jnp.float32),
            pltpu.VMEM((250, D), jnp.float32),
            pltpu.VMEM((BLK, D), jnp.float32),
        ],
        compiler_params=pltpu.CompilerParams(use_tc_tiling_on_sc=False),
    )(h)


def _tc_dense_body(act, msg_ref, h_ref, w_ref, b_ref, o_ref):
    gi = pl.program_id(0) * TC_R + lax.broadcasted_iota(jnp.int32, (TC_R, 1), 0)
    third = jnp.float32(1.0 / 3.0)
    inv = jnp.where(
        gi >= 50000, jnp.float32(0.5),
        jnp.where((gi == 0) | (gi == 49999), third, jnp.float32(0.25)))
    u = (msg_ref[...] + h_ref[...]) * inv
    y = jnp.dot(u, w_ref[...], preferred_element_type=jnp.float32) + b_ref[...]
    o_ref[...] = jnp.maximum(y, 0.0) if act else y


@functools.partial(jax.jit, static_argnames=("act",))
def _tc_dense(msg, h, w, b, act):
    grid = N // TC_R
    return pl.pallas_call(
        functools.partial(_tc_dense_body, act),
        grid=(grid,),
        in_specs=[
            pl.BlockSpec((TC_R, D), lambda i: (i, 0)),
            pl.BlockSpec((TC_R, D), lambda i: (i, 0)),
            pl.BlockSpec((D, D), lambda i: (0, 0)),
            pl.BlockSpec((1, D), lambda i: (0, 0)),
        ],
        out_specs=pl.BlockSpec((TC_R, D), lambda i: (i, 0)),
        out_shape=jax.ShapeDtypeStruct((N, D), jnp.float32),
    )(msg, h, w, b)


def kernel(x, W1, b1, W2, b2):
    b1r = b1.reshape(1, D)
    b2r = b2.reshape(1, D)
    msg1 = _sc_agg(x)
    h1 = _tc_dense(msg1, x, W1, b1r, act=True)
    msg2 = _sc_agg(h1)
    return _tc_dense(msg2, h1, W2, b2r, act=False)


# speedup vs baseline: 5.9119x; 1.1717x over previous
import functools

import jax
import jax.numpy as jnp
from jax import lax
from jax.experimental import pallas as pl
from jax.experimental.pallas import tpu as pltpu
from jax.experimental.pallas import tpu_sc as plsc

N = 100000
D = 256
LANES = 16
NCHUNK = D // LANES

BLK = 50
NPAR = 26
NCH = 100
TREE_BLKS = 50000 // BLK
LEAF_BLKS = 50000 // BLK
NWORKERS = 32
LAST_TREE_R0 = 50000 - BLK
ZROW = 27

TC_R = 1000


def _sc_agg_body(h_hbm, out_hbm, par_v, ch_v, out_v, sem_par, sem_ch, sem_out):
    wid = lax.axis_index("s") * 2 + lax.axis_index("c")

    def zero_row(buf, row):
        for c in range(NCHUNK):
            par_v[buf, row, pl.ds(c * LANES, LANES)] = jnp.zeros(
                (LANES,), jnp.float32)

    zero_row(0, ZROW)
    zero_row(1, ZROW)

    def tree_bi(t):
        last = wid + NWORKERS * 30 + jnp.where(wid < TREE_BLKS % NWORKERS,
                                               NWORKERS, 0)
        return jnp.minimum(t * NWORKERS + wid, last)

    def leaf_bi(t):
        last = wid + NWORKERS * 30 + jnp.where(wid < LEAF_BLKS % NWORKERS,
                                               NWORKERS, 0)
        return TREE_BLKS + jnp.minimum(t * NWORKERS + wid, last)

    def in_copies(bi, b, with_ch):
        r0 = bi * BLK
        p0 = jnp.maximum((r0 - 1) >> 1, 0)
        cps = [pltpu.make_async_copy(h_hbm.at[pl.ds(p0, NPAR)],
                                     par_v.at[b, pl.ds(0, NPAR)], sem_par)]
        if with_ch:
            c0 = 2 * r0 + 1
            last_src = jnp.minimum(c0 + NCH - 1, N - 1)
            cps.append(pltpu.make_async_copy(
                h_hbm.at[pl.ds(c0, NCH - 1)],
                ch_v.at[b, pl.ds(0, NCH - 1)], sem_ch))
            cps.append(pltpu.make_async_copy(
                h_hbm.at[pl.ds(last_src, 1)],
                ch_v.at[b, pl.ds(NCH - 1, 1)], sem_ch))
        return cps

    def out_copy(bi, b):
        return pltpu.make_async_copy(
            out_v.at[b], out_hbm.at[pl.ds(bi * BLK, BLK)], sem_out)

    def pipeline(bi_of, n, with_ch, compute):
        def fire(t):
            for cp in in_copies(bi_of(t), t & 1, with_ch):
                cp.start()

        def wait_in(t):
            for cp in in_copies(bi_of(t), t & 1, with_ch):
                cp.wait()

        fire(0)

        def body(t, _):
            wait_in(t)

            @pl.when(t + 1 < n)
            def _():
                fire(t + 1)

            @pl.when(t >= 2)
            def _():
                out_copy(bi_of(t - 2), t & 1).wait()

            compute(bi_of(t), t & 1)
            out_copy(bi_of(t), t & 1).start()
            return 0

        lax.fori_loop(0, n, body, 0)
        out_copy(bi_of(n - 2), n & 1).wait()
        out_copy(bi_of(n - 1), (n - 1) & 1).wait()

    def tree_compute(bi, b):
        r0 = bi * BLK
        p0 = jnp.maximum((r0 - 1) >> 1, 0)
        scale = jnp.where(r0 == LAST_TREE_R0, 0.0, 1.0).astype(jnp.float32)
        for c in range(NCHUNK):
            sl = pl.ds(c * LANES, LANES)
            ch_v[b, NCH - 1, sl] = ch_v[b, NCH - 1, sl] * scale

        def node(k, _):
            g = r0 + k
            pk = jnp.where(g == 0, ZROW, ((g - 1) >> 1) - p0)
            for c in range(NCHUNK):
                sl = pl.ds(c * LANES, LANES)
                out_v[b, k, sl] = (par_v[b, pk, sl] + ch_v[b, 2 * k, sl]
                                   + ch_v[b, 2 * k + 1, sl])
            return 0

        lax.fori_loop(0, BLK, node, 0)

    def leaf_compute(bi, b):
        r0 = bi * BLK
        p0 = (r0 - 1) >> 1

        def node(k, _):
            pk = ((r0 + k - 1) >> 1) - p0
            for c in range(NCHUNK):
                sl = pl.ds(c * LANES, LANES)
                out_v[b, k, sl] = par_v[b, pk, sl]
            return 0

        lax.fori_loop(0, BLK, node, 0)

    n_tree = (TREE_BLKS // NWORKERS) + jnp.where(
        wid < TREE_BLKS % NWORKERS, 1, 0)
    n_leaf = (LEAF_BLKS // NWORKERS) + jnp.where(
        wid < LEAF_BLKS % NWORKERS, 1, 0)
    pipeline(tree_bi, n_tree, True, tree_compute)
    pipeline(leaf_bi, n_leaf, False, leaf_compute)


@jax.jit
def _sc_agg(h):
    mesh = plsc.VectorSubcoreMesh(core_axis_name="c", subcore_axis_name="s")
    return pl.kernel(
        _sc_agg_body,
        out_type=jax.ShapeDtypeStruct((N, D), jnp.float32),
        mesh=mesh,
        scratch_types=[
            pltpu.VMEM((2, ZROW + 1, D), jnp.float32),
            pltpu.VMEM((2, NCH, D), jnp.float32),
            pltpu.VMEM((2, BLK, D), jnp.float32),
            pltpu.SemaphoreType.DMA,
            pltpu.SemaphoreType.DMA,
            pltpu.SemaphoreType.DMA,
        ],
        compiler_params=pltpu.CompilerParams(use_tc_tiling_on_sc=False),
    )(h)


def _tc_dense_body(act, msg_ref, h_ref, w_ref, b_ref, o_ref):
    gi = pl.program_id(0) * TC_R + lax.broadcasted_iota(jnp.int32, (TC_R, 1), 0)
    third = jnp.float32(1.0 / 3.0)
    inv = jnp.where(
        gi >= 50000, jnp.float32(0.5),
        jnp.where((gi == 0) | (gi == 49999), third, jnp.float32(0.25)))
    u = (msg_ref[...] + h_ref[...]) * inv
    y = jnp.dot(u, w_ref[...], preferred_element_type=jnp.float32) + b_ref[...]
    o_ref[...] = jnp.maximum(y, 0.0) if act else y


@functools.partial(jax.jit, static_argnames=("act",))
def _tc_dense(msg, h, w, b, act):
    grid = N // TC_R
    return pl.pallas_call(
        functools.partial(_tc_dense_body, act),
        grid=(grid,),
        in_specs=[
            pl.BlockSpec((TC_R, D), lambda i: (i, 0)),
            pl.BlockSpec((TC_R, D), lambda i: (i, 0)),
            pl.BlockSpec((D, D), lambda i: (0, 0)),
            pl.BlockSpec((1, D), lambda i: (0, 0)),
        ],
        out_specs=pl.BlockSpec((TC_R, D), lambda i: (i, 0)),
        out_shape=jax.ShapeDtypeStruct((N, D), jnp.float32),
    )(msg, h, w, b)


def kernel(x, W1, b1, W2, b2):
    b1r = b1.reshape(1, D)
    b2r = b2.reshape(1, D)
    msg1 = _sc_agg(x)
    h1 = _tc_dense(msg1, x, W1, b1r, act=True)
    msg2 = _sc_agg(h1)
    return _tc_dense(msg2, h1, W2, b2r, act=False)


# speedup vs baseline: 8.2703x; 1.3989x over previous
import functools

import jax
import jax.numpy as jnp
from jax import lax
from jax.experimental import pallas as pl
from jax.experimental.pallas import tpu as pltpu
from jax.experimental.pallas import tpu_sc as plsc

N = 100000
D = 256
LANES = 16
NCHUNK = D // LANES

BLK = 64
NPAR = 48
NCH = 136
TREE_BLKS = 782
NBLK = 1563
NWORKERS = 32

TC_R = 1000


def _sc_agg_body(h_hbm, out_hbm, par_v, ch_v, out_v, sem_par, sem_ch, sem_out):
    wid = lax.axis_index("s") * 2 + lax.axis_index("c")

    def tree_bi(t):
        return t * NWORKERS + wid

    def leaf_bi(t):
        return TREE_BLKS + t * NWORKERS + wid

    def block_r0(bi):
        return jnp.minimum(bi * BLK, N - BLK)

    def in_copies(bi, b, with_ch):
        r0 = block_r0(bi)
        p0 = pl.multiple_of(jnp.maximum((r0 >> 1) - 8, 0), 8)
        cps = [pltpu.make_async_copy(h_hbm.at[pl.ds(p0, NPAR)],
                                     par_v.at[b, pl.ds(0, NPAR)], sem_par)]
        if with_ch:
            c0 = pl.multiple_of(jnp.minimum(2 * r0, N - NCH), 8)
            cps.append(pltpu.make_async_copy(
                h_hbm.at[pl.ds(c0, NCH)], ch_v.at[b, pl.ds(0, NCH)], sem_ch))
        return cps

    def out_copy(bi, b):
        return pltpu.make_async_copy(
            out_v.at[b],
            out_hbm.at[pl.ds(pl.multiple_of(block_r0(bi), 8), BLK)], sem_out)

    def pipeline(bi_of, n, with_ch, compute):
        def fire(t):
            for cp in in_copies(bi_of(t), t & 1, with_ch):
                cp.start()

        def wait_in(t):
            for cp in in_copies(bi_of(t), t & 1, with_ch):
                cp.wait()

        fire(0)

        def body(t, _):
            wait_in(t)

            @pl.when(t + 1 < n)
            def _():
                fire(t + 1)

            @pl.when(t >= 2)
            def _():
                out_copy(bi_of(t - 2), t & 1).wait()

            compute(bi_of(t), t & 1)
            out_copy(bi_of(t), t & 1).start()
            return 0

        lax.fori_loop(0, n, body, 0)
        out_copy(bi_of(n - 2), n & 1).wait()
        out_copy(bi_of(n - 1), (n - 1) & 1).wait()

    def tree_compute(bi, b):
        r0 = block_r0(bi)
        p0 = jnp.maximum((r0 >> 1) - 8, 0)
        ch_off = 2 * r0 - jnp.minimum(2 * r0, N - NCH)

        def node(k, _):
            g = r0 + k
            pk = jnp.maximum(((g - 1) >> 1) - p0, 0)
            i1 = jnp.minimum(2 * k + 1 + ch_off, NCH - 1)
            i2 = jnp.minimum(2 * k + 2 + ch_off, NCH - 1)
            mp = jnp.where(g > 0, 1.0, 0.0).astype(jnp.float32)
            m1 = jnp.where(2 * g + 1 < N, 1.0, 0.0).astype(jnp.float32)
            m2 = jnp.where(2 * g + 2 < N, 1.0, 0.0).astype(jnp.float32)
            for c in range(NCHUNK):
                sl = pl.ds(c * LANES, LANES)
                out_v[b, k, sl] = (mp * par_v[b, pk, sl]
                                   + m1 * ch_v[b, i1, sl]
                                   + m2 * ch_v[b, i2, sl])
            return 0

        lax.fori_loop(0, BLK, node, 0)

    def leaf_compute(bi, b):
        r0 = block_r0(bi)
        p0 = (r0 >> 1) - 8

        def node(k, _):
            pk = ((r0 + k - 1) >> 1) - p0
            for c in range(NCHUNK):
                sl = pl.ds(c * LANES, LANES)
                out_v[b, k, sl] = par_v[b, pk, sl]
            return 0

        lax.fori_loop(0, BLK, node, 0)

    n_tree = (TREE_BLKS // NWORKERS) + jnp.where(
        wid < TREE_BLKS % NWORKERS, 1, 0)
    n_leaf = ((NBLK - TREE_BLKS) // NWORKERS) + jnp.where(
        wid < (NBLK - TREE_BLKS) % NWORKERS, 1, 0)
    pipeline(tree_bi, n_tree, True, tree_compute)
    pipeline(leaf_bi, n_leaf, False, leaf_compute)


@jax.jit
def _sc_agg(h):
    mesh = plsc.VectorSubcoreMesh(core_axis_name="c", subcore_axis_name="s")
    return pl.kernel(
        _sc_agg_body,
        out_type=jax.ShapeDtypeStruct((N, D), jnp.float32),
        mesh=mesh,
        scratch_types=[
            pltpu.VMEM((2, NPAR, D), jnp.float32),
            pltpu.VMEM((2, NCH, D), jnp.float32),
            pltpu.VMEM((2, BLK, D), jnp.float32),
            pltpu.SemaphoreType.DMA,
            pltpu.SemaphoreType.DMA,
            pltpu.SemaphoreType.DMA,
        ],
    )(h)


def _tc_dense_body(act, msg_ref, h_ref, w_ref, b_ref, o_ref):
    gi = pl.program_id(0) * TC_R + lax.broadcasted_iota(jnp.int32, (TC_R, 1), 0)
    third = jnp.float32(1.0 / 3.0)
    inv = jnp.where(
        gi >= 50000, jnp.float32(0.5),
        jnp.where((gi == 0) | (gi == 49999), third, jnp.float32(0.25)))
    u = (msg_ref[...] + h_ref[...]) * inv
    y = jnp.dot(u, w_ref[...], preferred_element_type=jnp.float32) + b_ref[...]
    o_ref[...] = jnp.maximum(y, 0.0) if act else y


@functools.partial(jax.jit, static_argnames=("act",))
def _tc_dense(msg, h, w, b, act):
    grid = N // TC_R
    return pl.pallas_call(
        functools.partial(_tc_dense_body, act),
        grid=(grid,),
        in_specs=[
            pl.BlockSpec((TC_R, D), lambda i: (i, 0)),
            pl.BlockSpec((TC_R, D), lambda i: (i, 0)),
            pl.BlockSpec((D, D), lambda i: (0, 0)),
            pl.BlockSpec((1, D), lambda i: (0, 0)),
        ],
        out_specs=pl.BlockSpec((TC_R, D), lambda i: (i, 0)),
        out_shape=jax.ShapeDtypeStruct((N, D), jnp.float32),
    )(msg, h, w, b)


def kernel(x, W1, b1, W2, b2):
    b1r = b1.reshape(1, D)
    b2r = b2.reshape(1, D)
    msg1 = _sc_agg(x)
    h1 = _tc_dense(msg1, x, W1, b1r, act=True)
    msg2 = _sc_agg(h1)
    return _tc_dense(msg2, h1, W2, b2r, act=False)


# speedup vs baseline: 8.8522x; 1.0704x over previous
import functools

import jax
import jax.numpy as jnp
from jax import lax
from jax.experimental import pallas as pl
from jax.experimental.pallas import tpu as pltpu
from jax.experimental.pallas import tpu_sc as plsc

N = 100000
D = 256
LANES = 16
NCHUNK = D // LANES

BLK = 64
NPAR = 40
NCH = 136
TREE_BLKS = 782
NBLK = 1563
NWORKERS = 32

TC_R = 2000


def _sc_agg_body(h_hbm, out_hbm, par_v, ch_v, out_v, sem_par, sem_ch, sem_out):
    wid = lax.axis_index("s") * 2 + lax.axis_index("c")

    def tree_bi(t):
        return t * NWORKERS + wid

    def leaf_bi(t):
        return TREE_BLKS + t * NWORKERS + wid

    def block_r0(bi):
        return jnp.minimum(bi * BLK, N - BLK)

    def in_copies(bi, b, with_ch):
        r0 = block_r0(bi)
        p0 = pl.multiple_of(jnp.maximum((r0 >> 1) - 8, 0), 8)
        cps = [pltpu.make_async_copy(h_hbm.at[pl.ds(p0, NPAR)],
                                     par_v.at[b, pl.ds(0, NPAR)], sem_par)]
        if with_ch:
            c0 = pl.multiple_of(jnp.minimum(2 * r0, N - NCH), 8)
            cps.append(pltpu.make_async_copy(
                h_hbm.at[pl.ds(c0, NCH)], ch_v.at[b, pl.ds(0, NCH)], sem_ch))
        return cps

    def out_copy(bi, b):
        return pltpu.make_async_copy(
            out_v.at[b],
            out_hbm.at[pl.ds(pl.multiple_of(block_r0(bi), 8), BLK)], sem_out)

    def pipeline(bi_of, n, with_ch, compute):
        def fire(t):
            for cp in in_copies(bi_of(t), t & 1, with_ch):
                cp.start()

        def wait_in(t):
            for cp in in_copies(bi_of(t), t & 1, with_ch):
                cp.wait()

        fire(0)

        def body(t, _):
            wait_in(t)

            @pl.when(t + 1 < n)
            def _():
                fire(t + 1)

            @pl.when(t >= 2)
            def _():
                out_copy(bi_of(t - 2), t & 1).wait()

            compute(bi_of(t), t & 1)
            out_copy(bi_of(t), t & 1).start()
            return 0

        lax.fori_loop(0, n, body, 0)
        out_copy(bi_of(n - 2), n & 1).wait()
        out_copy(bi_of(n - 1), (n - 1) & 1).wait()

    def tree_compute(bi, b):
        r0 = block_r0(bi)
        p0 = jnp.maximum((r0 >> 1) - 8, 0)
        ch_off = 2 * r0 - jnp.minimum(2 * r0, N - NCH)

        def node(k, _):
            g = r0 + k
            pk = jnp.maximum(((g - 1) >> 1) - p0, 0)
            i1 = jnp.minimum(2 * k + 1 + ch_off, NCH - 1)
            i2 = jnp.minimum(2 * k + 2 + ch_off, NCH - 1)
            mp = jnp.where(g > 0, 1.0, 0.0).astype(jnp.float32)
            m1 = jnp.where(2 * g + 1 < N, 1.0, 0.0).astype(jnp.float32)
            m2 = jnp.where(2 * g + 2 < N, 1.0, 0.0).astype(jnp.float32)
            for c in range(NCHUNK):
                sl = pl.ds(c * LANES, LANES)
                out_v[b, k, sl] = (mp * par_v[b, pk, sl]
                                   + m1 * ch_v[b, i1, sl]
                                   + m2 * ch_v[b, i2, sl])
            return 0

        lax.fori_loop(0, BLK, node, 0)

    def leaf_compute(bi, b):
        r0 = block_r0(bi)
        p0 = (r0 >> 1) - 8

        def node(k, _):
            pk = ((r0 + k - 1) >> 1) - p0
            for c in range(NCHUNK):
                sl = pl.ds(c * LANES, LANES)
                out_v[b, k, sl] = par_v[b, pk, sl]
            return 0

        lax.fori_loop(0, BLK, node, 0)

    n_tree = (TREE_BLKS // NWORKERS) + jnp.where(
        wid < TREE_BLKS % NWORKERS, 1, 0)
    n_leaf = ((NBLK - TREE_BLKS) // NWORKERS) + jnp.where(
        wid < (NBLK - TREE_BLKS) % NWORKERS, 1, 0)
    pipeline(tree_bi, n_tree, True, tree_compute)
    pipeline(leaf_bi, n_leaf, False, leaf_compute)


@jax.jit
def _sc_agg(h):
    mesh = plsc.VectorSubcoreMesh(core_axis_name="c", subcore_axis_name="s")
    return pl.kernel(
        _sc_agg_body,
        out_type=jax.ShapeDtypeStruct((N, D), jnp.float32),
        mesh=mesh,
        scratch_types=[
            pltpu.VMEM((2, NPAR, D), jnp.float32),
            pltpu.VMEM((2, NCH, D), jnp.float32),
            pltpu.VMEM((2, BLK, D), jnp.float32),
            pltpu.SemaphoreType.DMA,
            pltpu.SemaphoreType.DMA,
            pltpu.SemaphoreType.DMA,
        ],
    )(h)


def _tc_dense_body(act, msg_ref, h_ref, w_ref, b_ref, o_ref):
    gi = pl.program_id(0) * TC_R + lax.broadcasted_iota(jnp.int32, (TC_R, 1), 0)
    third = jnp.float32(1.0 / 3.0)
    inv = jnp.where(
        gi >= 50000, jnp.float32(0.5),
        jnp.where((gi == 0) | (gi == 49999), third, jnp.float32(0.25)))
    u = (msg_ref[...] + h_ref[...]) * inv
    y = jnp.dot(u, w_ref[...], preferred_element_type=jnp.float32) + b_ref[...]
    o_ref[...] = jnp.maximum(y, 0.0) if act else y


@functools.partial(jax.jit, static_argnames=("act",))
def _tc_dense(msg, h, w, b, act):
    grid = N // TC_R
    return pl.pallas_call(
        functools.partial(_tc_dense_body, act),
        grid=(grid,),
        in_specs=[
            pl.BlockSpec((TC_R, D), lambda i: (i, 0)),
            pl.BlockSpec((TC_R, D), lambda i: (i, 0)),
            pl.BlockSpec((D, D), lambda i: (0, 0)),
            pl.BlockSpec((1, D), lambda i: (0, 0)),
        ],
        out_specs=pl.BlockSpec((TC_R, D), lambda i: (i, 0)),
        out_shape=jax.ShapeDtypeStruct((N, D), jnp.float32),
    )(msg, h, w, b)


def kernel(x, W1, b1, W2, b2):
    b1r = b1.reshape(1, D)
    b2r = b2.reshape(1, D)
    msg1 = _sc_agg(x)
    h1 = _tc_dense(msg1, x, W1, b1r, act=True)
    msg2 = _sc_agg(h1)
    return _tc_dense(msg2, h1, W2, b2r, act=False)
